# baseline (speedup 1.0000x reference)
import jax
import jax.numpy as jnp
from jax.experimental import pallas as pl


def _copy_body(u_ref, o_ref):
    o_ref[...] = u_ref[:, 0, 0, :]


def kernel(u):
    u4 = u.reshape(64, 4, 1, 32768)
    grid = 8
    return pl.pallas_call(
        _copy_body,
        grid=(grid,),
        in_specs=[pl.BlockSpec((64 // grid, 1, 1, 32768), lambda i: (i, 0, 0, 0))],
        out_specs=pl.BlockSpec((64 // grid, 32768), lambda i: (i, 0)),
        out_shape=jax.ShapeDtypeStruct((64, 32768), jnp.float32),
    )(u4)
